# Initial kernel scaffold; baseline (speedup 1.0000x reference)
import jax
import jax.numpy as jnp
from jax.experimental import pallas as pl


def kernel(tokens, weight):
    raise NotImplementedError("write your pallas kernel here")


# speedup vs baseline: 1.4769x; 1.4769x over previous
import functools

import jax
import jax.numpy as jnp
from jax import lax
from jax.experimental import pallas as pl
from jax.experimental.pallas import tpu as pltpu
from jax.experimental.pallas import tpu_sc as plsc

NUM_EMB = 1000000
DIM = 32
B_TOTAL = 4096 * 200
NC, NS = 2, 16
NW = NC * NS
B_PER_W = B_TOTAL // NW
IDX_MINOR = 128
K = 8
CHUNK = K * IDX_MINOR
NSTEPS = B_PER_W // CHUNK
N_IDX_ROWS = B_PER_W // IDX_MINOR

_mesh = plsc.VectorSubcoreMesh(
    core_axis_name="c", subcore_axis_name="s", num_cores=NC, num_subcores=NS
)


@functools.partial(
    pl.kernel,
    out_type=jax.ShapeDtypeStruct((B_TOTAL, DIM), jnp.float32),
    mesh=_mesh,
    compiler_params=pltpu.CompilerParams(use_tc_tiling_on_sc=False),
    scratch_types=[
        pltpu.VMEM((N_IDX_ROWS, IDX_MINOR), jnp.int32),
        pltpu.VMEM((CHUNK, DIM), jnp.float32),
        pltpu.SemaphoreType.DMA,
        pltpu.SemaphoreType.DMA,
    ],
)
def _emb_lookup(idx_hbm, table_hbm, out_hbm, idx_v, rows_v, gsem, wsem):
    wid = lax.axis_index("s") * NC + lax.axis_index("c")
    base = wid * B_PER_W
    pltpu.sync_copy(idx_hbm.at[wid], idx_v)

    @pl.loop(0, NSTEPS)
    def _step(i):
        cps = [
            pltpu.async_copy(
                table_hbm.at[idx_v.at[i * K + b]],
                rows_v.at[pl.ds(b * IDX_MINOR, IDX_MINOR)],
                gsem,
            )
            for b in range(K)
        ]
        for cp in cps:
            cp.wait()
        pltpu.async_copy(
            rows_v, out_hbm.at[pl.ds(base + i * CHUNK, CHUNK)], wsem
        ).wait()


def kernel(tokens, weight):
    idx = tokens.astype(jnp.int32).reshape(NW, N_IDX_ROWS, IDX_MINOR)
    out = _emb_lookup(idx, weight)
    return out.reshape(tokens.shape + (DIM,))


# speedup vs baseline: 1.5060x; 1.0197x over previous
import functools

import jax
import jax.numpy as jnp
from jax import lax
from jax.experimental import pallas as pl
from jax.experimental.pallas import tpu as pltpu
from jax.experimental.pallas import tpu_sc as plsc

NUM_EMB = 1000000
DIM = 32
B_TOTAL = 4096 * 200
NC, NS = 2, 16
NW = NC * NS
B_PER_W = B_TOTAL // NW
IDX_MINOR = 128
K = 10
CHUNK = K * IDX_MINOR
NSTEPS = B_PER_W // CHUNK
N_IDX_ROWS = B_PER_W // IDX_MINOR

_mesh = plsc.VectorSubcoreMesh(
    core_axis_name="c", subcore_axis_name="s", num_cores=NC, num_subcores=NS
)


@functools.partial(
    pl.kernel,
    out_type=jax.ShapeDtypeStruct((B_TOTAL, DIM), jnp.float32),
    mesh=_mesh,
    compiler_params=pltpu.CompilerParams(use_tc_tiling_on_sc=False),
    scratch_types=[
        pltpu.VMEM((N_IDX_ROWS, IDX_MINOR), jnp.int32),
        pltpu.VMEM((CHUNK, DIM), jnp.float32),
        pltpu.VMEM((CHUNK, DIM), jnp.float32),
        pltpu.SemaphoreType.DMA,
        pltpu.SemaphoreType.DMA,
        pltpu.SemaphoreType.DMA,
        pltpu.SemaphoreType.DMA,
    ],
)
def _emb_lookup(idx_hbm, table_hbm, out_hbm, idx_v, buf0, buf1,
                g0, g1, w0, w1):
    wid = lax.axis_index("s") * NC + lax.axis_index("c")
    base = wid * B_PER_W
    pltpu.sync_copy(idx_hbm.at[wid], idx_v)

    def fire_gathers(c, buf, sem):
        return [
            pltpu.async_copy(
                table_hbm.at[idx_v.at[c * K + b]],
                buf.at[pl.ds(b * IDX_MINOR, IDX_MINOR)],
                sem,
            )
            for b in range(K)
        ]

    def drain_gathers(c, buf, sem):
        for b in range(K):
            pltpu.make_async_copy(
                table_hbm.at[idx_v.at[c * K + b]],
                buf.at[pl.ds(b * IDX_MINOR, IDX_MINOR)],
                sem,
            ).wait()

    def writeback(c, buf, sem):
        pltpu.async_copy(
            buf, out_hbm.at[pl.ds(base + c * CHUNK, CHUNK)], sem
        ).wait()

    fire_gathers(0, buf0, g0)
    fire_gathers(1, buf1, g1)

    @pl.loop(0, NSTEPS // 2)
    def _pair(j):
        c0 = 2 * j
        drain_gathers(c0, buf0, g0)
        writeback(c0, buf0, w0)

        @pl.when(c0 + 2 < NSTEPS)
        def _():
            fire_gathers(c0 + 2, buf0, g0)

        drain_gathers(c0 + 1, buf1, g1)
        writeback(c0 + 1, buf1, w1)

        @pl.when(c0 + 3 < NSTEPS)
        def _():
            fire_gathers(c0 + 3, buf1, g1)


def kernel(tokens, weight):
    idx = tokens.astype(jnp.int32).reshape(NW, N_IDX_ROWS, IDX_MINOR)
    out = _emb_lookup(idx, weight)
    return out.reshape(tokens.shape + (DIM,))
